# baseline (speedup 1.0000x reference)
import functools

import jax
import jax.numpy as jnp
from jax import lax
from jax.experimental import pallas as pl
from jax.experimental.pallas import tpu as pltpu
from jax.experimental.pallas import tpu_sc as plsc

N = 10000
D = 128
H = 128
O = 64
E = 320000

NC = 2
NS = 16
NW = NC * NS

CHUNK = 112
NCHUNK = 92
NPAIR = NCHUNK // 2
NQUAD = NCHUNK // 4
EPW = NCHUNK * CHUNK
EPAD = NW * EPW

ROWS_PT = 632
NPAD = NS * ROWS_PT

BN = 1000


def _sc_mesh():
    return plsc.VectorSubcoreMesh(core_axis_name="c", subcore_axis_name="s")


@functools.partial(
    pl.kernel,
    out_type=jax.ShapeDtypeStruct((NC, NPAD, H), jnp.float32),
    mesh=_sc_mesh(),
    scratch_types=[
        pltpu.VMEM((EPW,), jnp.int32),
        pltpu.VMEM((CHUNK, H), jnp.float32),
        pltpu.VMEM_SHARED((NPAD, H), jnp.float32),
    ],
)
def _sc_degree(dste_hbm, ones_hbm, zeros_hbm, out_hbm, dst_all, ones_v,
               deg_sh):
    c = lax.axis_index("c")
    s = lax.axis_index("s")
    w = c * NS + s
    base = s * ROWS_PT
    pltpu.sync_copy(zeros_hbm, deg_sh.at[pl.ds(base, ROWS_PT)])
    pltpu.sync_copy(ones_hbm, ones_v)
    pltpu.sync_copy(dste_hbm.at[pl.ds(w * EPW, EPW)], dst_all)
    plsc.subcore_barrier()

    def step(j, carry):
        pltpu.sync_copy(
            ones_v, deg_sh.at[dst_all.at[pl.ds(j * CHUNK, CHUNK)]],
            add=True)
        return carry

    lax.fori_loop(0, NCHUNK, step, 0)
    plsc.subcore_barrier()
    pltpu.sync_copy(deg_sh.at[pl.ds(base, ROWS_PT)],
                    out_hbm.at[c, pl.ds(base, ROWS_PT)])


@functools.partial(
    pl.kernel,
    out_type=jax.ShapeDtypeStruct((NC, NPAD, H), jnp.float32),
    mesh=_sc_mesh(),
    scratch_types=[
        pltpu.VMEM((EPW,), jnp.int32),
        pltpu.VMEM((EPW,), jnp.int32),
        pltpu.VMEM((CHUNK, H), jnp.float32),
        pltpu.VMEM((CHUNK, H), jnp.float32),
        pltpu.VMEM_SHARED((NPAD, H), jnp.float32),
        pltpu.SemaphoreType.DMA,
        pltpu.SemaphoreType.DMA,
        pltpu.SemaphoreType.DMA,
        pltpu.SemaphoreType.DMA,
    ],
)
def _sc_spmm(xp_hbm, srce_hbm, dste_hbm, zeros_hbm, out_hbm,
             src_all, dst_all, buf_a, buf_b, y_sh,
             sga, sgb, ssa, ssb):
    c = lax.axis_index("c")
    s = lax.axis_index("s")
    w = c * NS + s
    base = s * ROWS_PT
    pltpu.sync_copy(zeros_hbm, y_sh.at[pl.ds(base, ROWS_PT)])
    pltpu.sync_copy(srce_hbm.at[pl.ds(w * EPW, EPW)], src_all)
    pltpu.sync_copy(dste_hbm.at[pl.ds(w * EPW, EPW)], dst_all)
    plsc.subcore_barrier()

    def _gather(j, buf, sem):
        pltpu.async_copy(
            xp_hbm.at[src_all.at[pl.ds(j * CHUNK, CHUNK)]], buf, sem)

    def _gather_wait(j, buf, sem):
        pltpu.make_async_copy(
            xp_hbm.at[src_all.at[pl.ds(j * CHUNK, CHUNK)]], buf, sem).wait()

    def _scatter(j, buf, sem):
        pltpu.async_copy(
            buf, y_sh.at[dst_all.at[pl.ds(j * CHUNK, CHUNK)]], sem,
            add=True)

    def _scatter_wait(j, buf, sem):
        pltpu.make_async_copy(
            buf, y_sh.at[dst_all.at[pl.ds(j * CHUNK, CHUNK)]], sem).wait()

    _gather(0, buf_a, sga)
    _gather(1, buf_b, sgb)

    def pair(i, carry):
        j = i * 2
        _gather_wait(j, buf_a, sga)
        _scatter(j, buf_a, ssa)
        _gather_wait(j + 1, buf_b, sgb)
        _scatter(j + 1, buf_b, ssb)

        @pl.when(i < NPAIR - 1)
        def _():
            _scatter_wait(j, buf_a, ssa)
            _gather(j + 2, buf_a, sga)
            _scatter_wait(j + 1, buf_b, ssb)
            _gather(j + 3, buf_b, sgb)

        return carry

    lax.fori_loop(0, NPAIR, pair, 0)
    jl = (NPAIR - 1) * 2
    _scatter_wait(jl, buf_a, ssa)
    _scatter_wait(jl + 1, buf_b, ssb)
    plsc.subcore_barrier()
    pltpu.sync_copy(y_sh.at[pl.ds(base, ROWS_PT)],
                    out_hbm.at[c, pl.ds(base, ROWS_PT)])


def _dis_from_deg(degp_ref):
    d0 = degp_ref[0, :, 0:1]
    d1 = degp_ref[1, :, 0:1]
    return lax.rsqrt(1.0 + d0 + d1)


def _tc_first_body(x_ref, w_ref, degp_ref, o_ref):
    t = jnp.dot(x_ref[...], w_ref[...], preferred_element_type=jnp.float32)
    o_ref[...] = t * _dis_from_deg(degp_ref)


def _tc_first(x, w, degp):
    return pl.pallas_call(
        _tc_first_body,
        grid=(N // BN,),
        in_specs=[pl.BlockSpec((BN, D), lambda i: (i, 0)),
                  pl.BlockSpec((D, H), lambda i: (0, 0)),
                  pl.BlockSpec((NC, BN, H), lambda i: (0, i, 0))],
        out_specs=pl.BlockSpec((BN, H), lambda i: (i, 0)),
        out_shape=jax.ShapeDtypeStruct((N, H), jnp.float32),
    )(x, w, degp)


def _tc_epi_body(y_ref, xp_ref, degp_ref, b_ref, g_ref, beta_ref, w_ref, o_ref):
    dis = _dis_from_deg(degp_ref)
    h = dis * (y_ref[0] + y_ref[1] + xp_ref[...]) + b_ref[...]
    r = jnp.maximum(h, 0.0)
    mu = jnp.mean(r, axis=-1, keepdims=True)
    cen = r - mu
    var = jnp.mean(cen * cen, axis=-1, keepdims=True)
    ln = cen * lax.rsqrt(var + 1e-5) * g_ref[...] + beta_ref[...]
    o_ref[...] = jnp.dot(ln, w_ref[...],
                         preferred_element_type=jnp.float32) * dis


def _tc_epi(yp, xp, degp, b, g, beta, w):
    return pl.pallas_call(
        _tc_epi_body,
        grid=(N // BN,),
        in_specs=[pl.BlockSpec((NC, BN, H), lambda i: (0, i, 0)),
                  pl.BlockSpec((BN, H), lambda i: (i, 0)),
                  pl.BlockSpec((NC, BN, H), lambda i: (0, i, 0)),
                  pl.BlockSpec((1, H), lambda i: (0, 0)),
                  pl.BlockSpec((1, H), lambda i: (0, 0)),
                  pl.BlockSpec((1, H), lambda i: (0, 0)),
                  pl.BlockSpec((H, H), lambda i: (0, 0))],
        out_specs=pl.BlockSpec((BN, H), lambda i: (i, 0)),
        out_shape=jax.ShapeDtypeStruct((N, H), jnp.float32),
    )(yp, xp, degp, b, g, beta, w)


def _tc_final_body(y_ref, xp_ref, degp_ref, b_ref, pw1_ref, pb1_ref,
                   pw2_ref, pb2_ref, emb_ref, ls_ref):
    dis = _dis_from_deg(degp_ref)
    h = dis * (y_ref[0] + y_ref[1] + xp_ref[...]) + b_ref[...]
    emb_ref[...] = h
    r = jnp.maximum(h, 0.0)
    t = jnp.dot(r, pw1_ref[...], preferred_element_type=jnp.float32) \
        + pb1_ref[...]
    u = jnp.dot(t, pw2_ref[...], preferred_element_type=jnp.float32) \
        + pb2_ref[...]
    m = jnp.max(u, axis=-1, keepdims=True)
    lse = jnp.log(jnp.sum(jnp.exp(u - m), axis=-1, keepdims=True)) + m
    ls_ref[...] = u - lse


def _tc_final(yp, xp, degp, b, pw1, pb1, pw2, pb2):
    return pl.pallas_call(
        _tc_final_body,
        grid=(N // BN,),
        in_specs=[pl.BlockSpec((NC, BN, H), lambda i: (0, i, 0)),
                  pl.BlockSpec((BN, H), lambda i: (i, 0)),
                  pl.BlockSpec((NC, BN, H), lambda i: (0, i, 0)),
                  pl.BlockSpec((1, H), lambda i: (0, 0)),
                  pl.BlockSpec((H, H), lambda i: (0, 0)),
                  pl.BlockSpec((1, H), lambda i: (0, 0)),
                  pl.BlockSpec((H, O), lambda i: (0, 0)),
                  pl.BlockSpec((1, O), lambda i: (0, 0))],
        out_specs=[pl.BlockSpec((BN, H), lambda i: (i, 0)),
                   pl.BlockSpec((BN, O), lambda i: (i, 0))],
        out_shape=[jax.ShapeDtypeStruct((N, H), jnp.float32),
                   jax.ShapeDtypeStruct((N, O), jnp.float32)],
    )(yp, xp, degp, b, pw1, pb1, pw2, pb2)


def kernel(x, edge_index, W1, b1, W2, b2, W3, b3,
           ln1_g, ln1_b, ln2_g, ln2_b, pW1, pb1, pW2, pb2):
    src = edge_index[0]
    dst = edge_index[1]
    npw = EPW - E // NW
    pad_src = (jnp.arange(npw, dtype=jnp.int32) * 57) % N
    pad_dst = N + (jnp.arange(npw, dtype=jnp.int32) % (NPAD - N))
    srce = jnp.concatenate(
        [src.reshape(NW, E // NW),
         jnp.broadcast_to(pad_src, (NW, npw))], axis=1).reshape(EPAD)
    dste = jnp.concatenate(
        [dst.reshape(NW, E // NW),
         jnp.broadcast_to(pad_dst, (NW, npw))], axis=1).reshape(EPAD)

    zeros_h = jnp.zeros((ROWS_PT, H), jnp.float32)
    ones_h = jnp.ones((CHUNK, H), jnp.float32)

    degp = _sc_degree(dste, ones_h, zeros_h)
    degp_n = degp[:, :N, :]

    x1p = _tc_first(x, W1, degp_n)
    y1 = _sc_spmm(x1p, srce, dste, zeros_h)[:, :N, :]
    x2p = _tc_epi(y1, x1p, degp_n, b1.reshape(1, H),
                  ln1_g.reshape(1, H), ln1_b.reshape(1, H), W2)
    y2 = _sc_spmm(x2p, srce, dste, zeros_h)[:, :N, :]
    x3p = _tc_epi(y2, x2p, degp_n, b2.reshape(1, H),
                  ln2_g.reshape(1, H), ln2_b.reshape(1, H), W3)
    y3 = _sc_spmm(x3p, srce, dste, zeros_h)[:, :N, :]
    emb, logsm = _tc_final(y3, x3p, degp_n, b3.reshape(1, H),
                           pW1, pb1.reshape(1, H), pW2, pb2.reshape(1, O))
    return (emb, logsm)


# speedup vs baseline: 1.2337x; 1.2337x over previous
import functools

import jax
import jax.numpy as jnp
from jax import lax
from jax.experimental import pallas as pl
from jax.experimental.pallas import tpu as pltpu
from jax.experimental.pallas import tpu_sc as plsc

N = 10000
D = 128
H = 128
O = 64
E = 320000

NC = 2
NS = 16
NW = NC * NS

CHUNK = 112
NCHUNK = 90
NPAIR = NCHUNK // 2
EPW = NCHUNK * CHUNK
EPAD = NW * EPW

ROWS_PT = 632
NPAD = NS * ROWS_PT

BN = 1000


def _sc_mesh():
    return plsc.VectorSubcoreMesh(core_axis_name="c", subcore_axis_name="s")


@functools.partial(
    pl.kernel,
    out_type=jax.ShapeDtypeStruct((NC, NPAD, H), jnp.float32),
    mesh=_sc_mesh(),
    scratch_types=[
        pltpu.VMEM((EPW,), jnp.int32),
        pltpu.VMEM((CHUNK, H), jnp.float32),
        pltpu.VMEM_SHARED((NPAD, H), jnp.float32),
    ],
)
def _sc_degree(dste_hbm, ones_hbm, zeros_hbm, out_hbm, dst_all, ones_v,
               deg_sh):
    c = lax.axis_index("c")
    s = lax.axis_index("s")
    w = c * NS + s
    base = s * ROWS_PT
    pltpu.sync_copy(zeros_hbm, deg_sh.at[pl.ds(base, ROWS_PT)])
    pltpu.sync_copy(ones_hbm, ones_v)
    pltpu.sync_copy(dste_hbm.at[pl.ds(w * EPW, EPW)], dst_all)
    plsc.subcore_barrier()

    def step(j, carry):
        pltpu.sync_copy(
            ones_v, deg_sh.at[dst_all.at[pl.ds(j * CHUNK, CHUNK)]],
            add=True)
        return carry

    lax.fori_loop(0, NCHUNK, step, 0)
    plsc.subcore_barrier()
    pltpu.sync_copy(deg_sh.at[pl.ds(base, ROWS_PT)],
                    out_hbm.at[c, pl.ds(base, ROWS_PT)])


@functools.partial(
    pl.kernel,
    out_type=jax.ShapeDtypeStruct((NC, NPAD, H), jnp.float32),
    mesh=_sc_mesh(),
    scratch_types=[
        pltpu.VMEM((EPW,), jnp.int32),
        pltpu.VMEM((EPW,), jnp.int32),
        pltpu.VMEM((CHUNK, H), jnp.float32),
        pltpu.VMEM((CHUNK, H), jnp.float32),
        pltpu.VMEM_SHARED((NPAD, H), jnp.float32),
        pltpu.SemaphoreType.DMA,
        pltpu.SemaphoreType.DMA,
    ],
)
def _sc_spmm(xp_hbm, srce_hbm, dste_hbm, zeros_hbm, out_hbm,
             src_all, dst_all, buf_a, buf_b, y_sh, sem_a, sem_b):
    c = lax.axis_index("c")
    s = lax.axis_index("s")
    w = c * NS + s
    base = s * ROWS_PT
    pltpu.sync_copy(zeros_hbm, y_sh.at[pl.ds(base, ROWS_PT)])
    pltpu.sync_copy(srce_hbm.at[pl.ds(w * EPW, EPW)], src_all)
    pltpu.sync_copy(dste_hbm.at[pl.ds(w * EPW, EPW)], dst_all)
    plsc.subcore_barrier()

    def _gather(j, buf, sem):
        pltpu.async_copy(
            xp_hbm.at[src_all.at[pl.ds(j * CHUNK, CHUNK)]], buf, sem)

    def _gather_wait(j, buf, sem):
        pltpu.make_async_copy(
            xp_hbm.at[src_all.at[pl.ds(j * CHUNK, CHUNK)]], buf, sem).wait()

    def _scatter(j, buf):
        pltpu.sync_copy(
            buf, y_sh.at[dst_all.at[pl.ds(j * CHUNK, CHUNK)]], add=True)

    _gather(0, buf_a, sem_a)

    def pair(i, carry):
        j = i * 2
        _gather(j + 1, buf_b, sem_b)
        _gather_wait(j, buf_a, sem_a)
        _scatter(j, buf_a)

        @pl.when(i < NPAIR - 1)
        def _():
            _gather(j + 2, buf_a, sem_a)

        _gather_wait(j + 1, buf_b, sem_b)
        _scatter(j + 1, buf_b)
        return carry

    lax.fori_loop(0, NPAIR, pair, 0)
    plsc.subcore_barrier()
    pltpu.sync_copy(y_sh.at[pl.ds(base, ROWS_PT)],
                    out_hbm.at[c, pl.ds(base, ROWS_PT)])


def _dis_from_deg(degp_ref):
    d0 = degp_ref[0, :, 0:1]
    d1 = degp_ref[1, :, 0:1]
    return lax.rsqrt(1.0 + d0 + d1)


def _tc_first_body(x_ref, w_ref, degp_ref, o_ref, dis_ref):
    t = jnp.dot(x_ref[...], w_ref[...], preferred_element_type=jnp.float32)
    dis = _dis_from_deg(degp_ref)
    dis_ref[...] = dis
    o_ref[...] = t * dis


def _tc_first(x, w, degp):
    return pl.pallas_call(
        _tc_first_body,
        grid=(N // BN,),
        in_specs=[pl.BlockSpec((BN, D), lambda i: (i, 0)),
                  pl.BlockSpec((D, H), lambda i: (0, 0)),
                  pl.BlockSpec((NC, BN, H), lambda i: (0, i, 0))],
        out_specs=[pl.BlockSpec((BN, H), lambda i: (i, 0)),
                   pl.BlockSpec((BN, 1), lambda i: (i, 0))],
        out_shape=[jax.ShapeDtypeStruct((N, H), jnp.float32),
                   jax.ShapeDtypeStruct((N, 1), jnp.float32)],
    )(x, w, degp)


def _tc_epi_body(y_ref, xp_ref, dis_ref, b_ref, g_ref, beta_ref, w_ref, o_ref):
    dis = dis_ref[...]
    h = dis * (y_ref[0] + y_ref[1] + xp_ref[...]) + b_ref[...]
    r = jnp.maximum(h, 0.0)
    mu = jnp.mean(r, axis=-1, keepdims=True)
    cen = r - mu
    var = jnp.mean(cen * cen, axis=-1, keepdims=True)
    ln = cen * lax.rsqrt(var + 1e-5) * g_ref[...] + beta_ref[...]
    o_ref[...] = jnp.dot(ln, w_ref[...],
                         preferred_element_type=jnp.float32) * dis


def _tc_epi(yp, xp, dis, b, g, beta, w):
    return pl.pallas_call(
        _tc_epi_body,
        grid=(N // BN,),
        in_specs=[pl.BlockSpec((NC, BN, H), lambda i: (0, i, 0)),
                  pl.BlockSpec((BN, H), lambda i: (i, 0)),
                  pl.BlockSpec((BN, 1), lambda i: (i, 0)),
                  pl.BlockSpec((1, H), lambda i: (0, 0)),
                  pl.BlockSpec((1, H), lambda i: (0, 0)),
                  pl.BlockSpec((1, H), lambda i: (0, 0)),
                  pl.BlockSpec((H, H), lambda i: (0, 0))],
        out_specs=pl.BlockSpec((BN, H), lambda i: (i, 0)),
        out_shape=jax.ShapeDtypeStruct((N, H), jnp.float32),
    )(yp, xp, dis, b, g, beta, w)


def _tc_final_body(y_ref, xp_ref, dis_ref, b_ref, pw1_ref, pb1_ref,
                   pw2_ref, pb2_ref, emb_ref, ls_ref):
    dis = dis_ref[...]
    h = dis * (y_ref[0] + y_ref[1] + xp_ref[...]) + b_ref[...]
    emb_ref[...] = h
    r = jnp.maximum(h, 0.0)
    t = jnp.dot(r, pw1_ref[...], preferred_element_type=jnp.float32) \
        + pb1_ref[...]
    u = jnp.dot(t, pw2_ref[...], preferred_element_type=jnp.float32) \
        + pb2_ref[...]
    m = jnp.max(u, axis=-1, keepdims=True)
    lse = jnp.log(jnp.sum(jnp.exp(u - m), axis=-1, keepdims=True)) + m
    ls_ref[...] = u - lse


def _tc_final(yp, xp, dis, b, pw1, pb1, pw2, pb2):
    return pl.pallas_call(
        _tc_final_body,
        grid=(N // BN,),
        in_specs=[pl.BlockSpec((NC, BN, H), lambda i: (0, i, 0)),
                  pl.BlockSpec((BN, H), lambda i: (i, 0)),
                  pl.BlockSpec((BN, 1), lambda i: (i, 0)),
                  pl.BlockSpec((1, H), lambda i: (0, 0)),
                  pl.BlockSpec((H, H), lambda i: (0, 0)),
                  pl.BlockSpec((1, H), lambda i: (0, 0)),
                  pl.BlockSpec((H, O), lambda i: (0, 0)),
                  pl.BlockSpec((1, O), lambda i: (0, 0))],
        out_specs=[pl.BlockSpec((BN, H), lambda i: (i, 0)),
                   pl.BlockSpec((BN, O), lambda i: (i, 0))],
        out_shape=[jax.ShapeDtypeStruct((N, H), jnp.float32),
                   jax.ShapeDtypeStruct((N, O), jnp.float32)],
    )(yp, xp, dis, b, pw1, pb1, pw2, pb2)


def kernel(x, edge_index, W1, b1, W2, b2, W3, b3,
           ln1_g, ln1_b, ln2_g, ln2_b, pW1, pb1, pW2, pb2):
    src = edge_index[0]
    dst = edge_index[1]
    npw = EPW - E // NW
    pad_src = (jnp.arange(npw, dtype=jnp.int32) * 57) % N
    pad_dst = N + (jnp.arange(npw, dtype=jnp.int32) % (NPAD - N))
    srce = jnp.concatenate(
        [src.reshape(NW, E // NW),
         jnp.broadcast_to(pad_src, (NW, npw))], axis=1).reshape(EPAD)
    dste = jnp.concatenate(
        [dst.reshape(NW, E // NW),
         jnp.broadcast_to(pad_dst, (NW, npw))], axis=1).reshape(EPAD)

    zeros_h = jnp.zeros((ROWS_PT, H), jnp.float32)
    ones_h = jnp.ones((CHUNK, H), jnp.float32)

    degp = _sc_degree(dste, ones_h, zeros_h)
    degp_n = degp[:, :N, :]

    x1p, dis = _tc_first(x, W1, degp_n)
    y1 = _sc_spmm(x1p, srce, dste, zeros_h)[:, :N, :]
    x2p = _tc_epi(y1, x1p, dis, b1.reshape(1, H),
                  ln1_g.reshape(1, H), ln1_b.reshape(1, H), W2)
    y2 = _sc_spmm(x2p, srce, dste, zeros_h)[:, :N, :]
    x3p = _tc_epi(y2, x2p, dis, b2.reshape(1, H),
                  ln2_g.reshape(1, H), ln2_b.reshape(1, H), W3)
    y3 = _sc_spmm(x3p, srce, dste, zeros_h)[:, :N, :]
    emb, logsm = _tc_final(y3, x3p, dis, b3.reshape(1, H),
                           pW1, pb1.reshape(1, H), pW2, pb2.reshape(1, O))
    return (emb, logsm)
